# Initial kernel scaffold; baseline (speedup 1.0000x reference)
import jax
import jax.numpy as jnp
from jax.experimental import pallas as pl


def kernel(inputs, W, edge_index, edge_vals):
    raise NotImplementedError("write your pallas kernel here")


# speedup vs baseline: 5.2709x; 5.2709x over previous
import functools

import jax
import jax.numpy as jnp
from jax import lax
from jax.experimental import pallas as pl
from jax.experimental.pallas import tpu as pltpu
from jax.experimental.pallas import tpu_sc as plsc

N_NODES = 10000
D = 128
N_EDGES = 320000

NC = 2
NS = 16
L = 16
NW = NC * NS

CHUNK = 128
CPW = ((N_EDGES + NW - 1) // NW + CHUNK - 1) // CHUNK
EPW = CPW * CHUNK
E_PAD = EPW * NW

TROWS = (N_NODES // NS) // 8 * 8
TAIL = N_NODES - NS * TROWS


def _sc_body(x_hbm, row_hbm, col_hbm, ev_hbm, out_hbm,
             colv, rowv, evv, rows_v, acc, sem):
    cid = lax.axis_index("c")
    sid = lax.axis_index("s")
    wid = sid * NC + cid

    def _agg():
        zeros16 = jnp.zeros((L,), jnp.float32)

        def _zero_row(i, carry):
            for v in range(D // L):
                rows_v[i, pl.ds(v * L, L)] = zeros16
            return carry

        lax.fori_loop(0, CHUNK, _zero_row, 0)
        base = sid * TROWS
        for k in range(TROWS // CHUNK):
            pltpu.sync_copy(rows_v, acc.at[pl.ds(base + k * CHUNK, CHUNK)])
        rem = TROWS - (TROWS // CHUNK) * CHUNK
        if rem:
            pltpu.sync_copy(rows_v.at[pl.ds(0, rem)],
                            acc.at[pl.ds(base + (TROWS // CHUNK) * CHUNK, rem)])

        @pl.when(sid == 0)
        def _zero_tail():
            pltpu.sync_copy(rows_v.at[pl.ds(0, TAIL)],
                            acc.at[pl.ds(NS * TROWS, TAIL)])

        plsc.subcore_barrier()

        pltpu.sync_copy(col_hbm.at[wid], colv)
        pltpu.sync_copy(row_hbm.at[wid], rowv)
        pltpu.sync_copy(ev_hbm.at[wid], evv)

        def _chunk(j, carry):
            pltpu.async_copy(x_hbm.at[colv.at[j]], rows_v, sem).wait()

            def _scale16(g, c2):
                evg = evv[j, pl.ds(g * L, L)]
                for e in range(L):
                    b = jnp.take_along_axis(evg, jnp.full((L,), e, jnp.int32),
                                            axis=0, mode="promise_in_bounds")
                    r = g * L + e
                    for v in range(D // L):
                        sl = pl.ds(v * L, L)
                        rows_v[r, sl] = rows_v[r, sl] * b
                return c2

            lax.fori_loop(0, CHUNK // L, _scale16, 0)

            pltpu.sync_copy(rows_v, acc.at[rowv.at[j]], add=True)
            return carry

        lax.fori_loop(0, CPW, _chunk, 0)
        plsc.subcore_barrier()

        pltpu.sync_copy(acc.at[pl.ds(base, TROWS)],
                        out_hbm.at[cid, pl.ds(base, TROWS)])

        @pl.when(sid == 0)
        def _out_tail():
            pltpu.sync_copy(acc.at[pl.ds(NS * TROWS, TAIL)],
                            out_hbm.at[cid, pl.ds(NS * TROWS, TAIL)])

    _agg()


@functools.cache
def _sc_agg():
    return pl.kernel(
        _sc_body,
        out_type=jax.ShapeDtypeStruct((NC, N_NODES, D), jnp.float32),
        mesh=plsc.VectorSubcoreMesh(core_axis_name="c", subcore_axis_name="s",
                                    num_cores=NC, num_subcores=NS),
        scratch_types=[
            pltpu.VMEM((CPW, CHUNK), jnp.int32),
            pltpu.VMEM((CPW, CHUNK), jnp.int32),
            pltpu.VMEM((CPW, CHUNK), jnp.float32),
            pltpu.VMEM((CHUNK, D), jnp.float32),
            pltpu.VMEM_SHARED((N_NODES, D), jnp.float32),
            pltpu.SemaphoreType.DMA,
        ],
    )


def _combine_body(p_ref, w_ref, o_ref):
    s = p_ref[0] + p_ref[1]
    o_ref[...] = jnp.maximum(
        jnp.dot(s, w_ref[...], preferred_element_type=jnp.float32), 0.0)


BM = 1000

_combine = pl.pallas_call(
    _combine_body,
    grid=(N_NODES // BM,),
    in_specs=[
        pl.BlockSpec((NC, BM, D), lambda i: (0, i, 0)),
        pl.BlockSpec((D, D), lambda i: (0, 0)),
    ],
    out_specs=pl.BlockSpec((BM, D), lambda i: (i, 0)),
    out_shape=jax.ShapeDtypeStruct((N_NODES, D), jnp.float32),
)


@jax.jit
def kernel(inputs, W, edge_index, edge_vals):
    row = edge_index[0]
    col = edge_index[1]
    pad = E_PAD - N_EDGES
    row_p = jnp.concatenate([row, jnp.zeros((pad,), jnp.int32)])
    col_p = jnp.concatenate([col, jnp.zeros((pad,), jnp.int32)])
    ev_p = jnp.concatenate([edge_vals, jnp.zeros((pad,), jnp.float32)])
    row2d = row_p.reshape(NW, CPW, CHUNK)
    col2d = col_p.reshape(NW, CPW, CHUNK)
    ev2d = ev_p.reshape(NW, CPW, CHUNK)
    partials = _sc_agg()(inputs, row2d, col2d, ev2d)
    return _combine(partials, W[0])
